# baseline (speedup 1.0000x reference)
import functools

import numpy as np

import jax
import jax.numpy as jnp
from jax import lax
from jax.experimental import pallas as pl
from jax.experimental.pallas import tpu as pltpu
from jax.experimental.pallas import tpu_sc as plsc

N_NODES = 10000
N_PAD = 10240
NC, NS = 2, 16
NW = NC * NS
DUMP_ROW = N_NODES + 64
ROW_BLK = 1024


def _tc_matmul(x, w):
    n, k = x.shape
    d = w.shape[1]

    def body(x_ref, w_ref, o_ref):
        o_ref[...] = jnp.dot(x_ref[...], w_ref[...],
                             preferred_element_type=jnp.float32)

    return pl.pallas_call(
        body,
        grid=(n // ROW_BLK,),
        in_specs=[
            pl.BlockSpec((ROW_BLK, k), lambda i: (i, 0)),
            pl.BlockSpec((k, d), lambda i: (0, 0)),
        ],
        out_specs=pl.BlockSpec((ROW_BLK, d), lambda i: (i, 0)),
        out_shape=jax.ShapeDtypeStruct((n, d), jnp.float32),
    )(x, w)


def _tc_combine(x, w_self, parts, b, relu, w_next=None):
    n, k = x.shape
    d = w_self.shape[1]
    d2 = None if w_next is None else w_next.shape[1]

    def body(x_ref, ws_ref, p_ref, b_ref, *rest):
        h = jnp.dot(x_ref[...], ws_ref[...], preferred_element_type=jnp.float32)
        h = h + p_ref[0] + p_ref[1] + b_ref[...]
        if relu:
            h = jnp.maximum(h, 0.0)
        if w_next is None:
            (o_ref,) = rest
            o_ref[...] = h
        else:
            wn_ref, h_ref, y_ref = rest
            h_ref[...] = h
            y_ref[...] = jnp.dot(h, wn_ref[...], preferred_element_type=jnp.float32)

    in_specs = [
        pl.BlockSpec((ROW_BLK, k), lambda i: (i, 0)),
        pl.BlockSpec((k, d), lambda i: (0, 0)),
        pl.BlockSpec((NC, ROW_BLK, d), lambda i: (0, i, 0)),
        pl.BlockSpec((1, d), lambda i: (0, 0)),
    ]
    operands = [x, w_self, parts, b.reshape(1, d)]
    if w_next is None:
        out_specs = pl.BlockSpec((ROW_BLK, d), lambda i: (i, 0))
        out_shape = jax.ShapeDtypeStruct((n, d), jnp.float32)
    else:
        in_specs.append(pl.BlockSpec((d, d2), lambda i: (0, 0)))
        operands.append(w_next)
        out_specs = [
            pl.BlockSpec((ROW_BLK, d), lambda i: (i, 0)),
            pl.BlockSpec((ROW_BLK, d2), lambda i: (i, 0)),
        ]
        out_shape = [
            jax.ShapeDtypeStruct((n, d), jnp.float32),
            jax.ShapeDtypeStruct((n, d2), jnp.float32),
        ]

    return pl.pallas_call(
        body,
        grid=(n // ROW_BLK,),
        in_specs=in_specs,
        out_specs=out_specs,
        out_shape=out_shape,
    )(*operands)


def _sc_agg(y, src3, dst3, zeros, n_j, edge_b):
    d = y.shape[1]
    rows_per_s = N_PAD // NS
    mesh = plsc.VectorSubcoreMesh(core_axis_name="c", subcore_axis_name="s")

    @functools.partial(
        pl.kernel,
        out_type=jax.ShapeDtypeStruct((NC, N_PAD, d), jnp.float32),
        mesh=mesh,
        scratch_types=[
            pltpu.VMEM_SHARED((N_PAD, d), jnp.float32),
            pltpu.VMEM((n_j, edge_b), jnp.int32),
            pltpu.VMEM((n_j, edge_b), jnp.int32),
            pltpu.VMEM((edge_b, d), jnp.float32),
            pltpu.SemaphoreType.DMA,
        ],
        compiler_params=pltpu.CompilerParams(use_tc_tiling_on_sc=False),
    )
    def k(y_hbm, src_hbm, dst_hbm, z_hbm, out_hbm, acc, src_v, dst_v,
          rows_v, sem):
        c = lax.axis_index("c")
        s = lax.axis_index("s")
        w = s * NC + c
        r0 = s * rows_per_s
        pltpu.sync_copy(z_hbm.at[pl.ds(r0, rows_per_s)],
                        acc.at[pl.ds(r0, rows_per_s)])
        pltpu.sync_copy(src_hbm.at[w], src_v)
        pltpu.sync_copy(dst_hbm.at[w], dst_v)
        plsc.subcore_barrier()

        @pl.loop(0, n_j)
        def edge_chunk(j):
            pltpu.async_copy(y_hbm.at[src_v.at[j]], rows_v, sem).wait()
            pltpu.sync_copy(rows_v, acc.at[dst_v.at[j]], add=True)

        plsc.subcore_barrier()
        pltpu.sync_copy(acc.at[pl.ds(r0, rows_per_s)],
                        out_hbm.at[c, pl.ds(r0, rows_per_s)])

    return k(y, src3, dst3, zeros)


EDGE_B0 = 128
EDGE_B1 = 128


def kernel(x, edge_index, W_self0, W_neigh0, b0, W_self1, W_neigh1, b1):
    n_edges = edge_index.shape[1]
    per_w = -(-n_edges // (NW * 2 * EDGE_B1)) * 2 * EDGE_B1
    e_pad = NW * per_w

    x_pad = jnp.zeros((N_PAD, x.shape[1]), jnp.float32).at[:N_NODES].set(x)
    n_tail = e_pad - n_edges
    tail_src = np.zeros((n_tail,), np.int32)
    tail_dst = N_NODES + (np.arange(n_tail, dtype=np.int32) % (N_PAD - N_NODES))
    src = jnp.concatenate([edge_index[0], tail_src])
    dst = jnp.concatenate([edge_index[1], tail_dst])
    src0 = src.reshape(NW, per_w // EDGE_B0, EDGE_B0)
    dst0 = dst.reshape(NW, per_w // EDGE_B0, EDGE_B0)
    src1 = src.reshape(NW, per_w // EDGE_B1, EDGE_B1)
    dst1 = dst.reshape(NW, per_w // EDGE_B1, EDGE_B1)

    d_hid = W_neigh0.shape[1]
    d_out = W_neigh1.shape[1]
    zeros_hid = jnp.zeros((N_PAD, d_hid), jnp.float32)
    zeros_out = jnp.zeros((N_PAD, d_out), jnp.float32)

    y0 = _tc_matmul(x_pad, W_neigh0)
    p0 = _sc_agg(y0, src0, dst0, zeros_hid, per_w // EDGE_B0, EDGE_B0)
    h, y1 = _tc_combine(x_pad, W_self0, p0, b0, relu=True, w_next=W_neigh1)
    p1 = _sc_agg(y1, src1, dst1, zeros_out, per_w // EDGE_B1, EDGE_B1)
    out = _tc_combine(h, W_self1, p1, b1, relu=False)
    return out[:N_NODES]


# speedup vs baseline: 1.4911x; 1.4911x over previous
import functools

import numpy as np

import jax
import jax.numpy as jnp
from jax import lax
from jax.experimental import pallas as pl
from jax.experimental.pallas import tpu as pltpu
from jax.experimental.pallas import tpu_sc as plsc

N_NODES = 10000
N_PAD = 10240
NC, NS = 2, 16
NW = NC * NS
DUMP_ROW = N_NODES + 64
ROW_BLK = 1024


def _tc_matmul(x, w):
    n, k = x.shape
    d = w.shape[1]

    def body(x_ref, w_ref, o_ref):
        o_ref[...] = jnp.dot(x_ref[...], w_ref[...],
                             preferred_element_type=jnp.float32)

    return pl.pallas_call(
        body,
        grid=(n // ROW_BLK,),
        in_specs=[
            pl.BlockSpec((ROW_BLK, k), lambda i: (i, 0)),
            pl.BlockSpec((k, d), lambda i: (0, 0)),
        ],
        out_specs=pl.BlockSpec((ROW_BLK, d), lambda i: (i, 0)),
        out_shape=jax.ShapeDtypeStruct((n, d), jnp.float32),
    )(x, w)


def _tc_combine(x, w_self, parts, b, relu, w_next=None):
    n, k = x.shape
    d = w_self.shape[1]
    d2 = None if w_next is None else w_next.shape[1]

    def body(x_ref, ws_ref, p_ref, b_ref, *rest):
        h = jnp.dot(x_ref[...], ws_ref[...], preferred_element_type=jnp.float32)
        h = h + p_ref[0] + p_ref[1] + b_ref[...]
        if relu:
            h = jnp.maximum(h, 0.0)
        if w_next is None:
            (o_ref,) = rest
            o_ref[...] = h
        else:
            wn_ref, h_ref, y_ref = rest
            h_ref[...] = h
            y_ref[...] = jnp.dot(h, wn_ref[...], preferred_element_type=jnp.float32)

    in_specs = [
        pl.BlockSpec((ROW_BLK, k), lambda i: (i, 0)),
        pl.BlockSpec((k, d), lambda i: (0, 0)),
        pl.BlockSpec((NC, ROW_BLK, d), lambda i: (0, i, 0)),
        pl.BlockSpec((1, d), lambda i: (0, 0)),
    ]
    operands = [x, w_self, parts, b.reshape(1, d)]
    if w_next is None:
        out_specs = pl.BlockSpec((ROW_BLK, d), lambda i: (i, 0))
        out_shape = jax.ShapeDtypeStruct((n, d), jnp.float32)
    else:
        in_specs.append(pl.BlockSpec((d, d2), lambda i: (0, 0)))
        operands.append(w_next)
        out_specs = [
            pl.BlockSpec((ROW_BLK, d), lambda i: (i, 0)),
            pl.BlockSpec((ROW_BLK, d2), lambda i: (i, 0)),
        ]
        out_shape = [
            jax.ShapeDtypeStruct((n, d), jnp.float32),
            jax.ShapeDtypeStruct((n, d2), jnp.float32),
        ]

    return pl.pallas_call(
        body,
        grid=(n // ROW_BLK,),
        in_specs=in_specs,
        out_specs=out_specs,
        out_shape=out_shape,
    )(*operands)


def _sc_agg(y, src3, dst3, zeros, n_j, edge_b):
    d = y.shape[1]
    rows_per_s = N_PAD // NS
    mesh = plsc.VectorSubcoreMesh(core_axis_name="c", subcore_axis_name="s")

    @functools.partial(
        pl.kernel,
        out_type=jax.ShapeDtypeStruct((NC, N_PAD, d), jnp.float32),
        mesh=mesh,
        scratch_types=[
            pltpu.VMEM_SHARED((N_PAD, d), jnp.float32),
            pltpu.VMEM((n_j, edge_b), jnp.int32),
            pltpu.VMEM((n_j, edge_b), jnp.int32),
            pltpu.VMEM((edge_b, d), jnp.float32),
            pltpu.SemaphoreType.DMA,
        ],
        compiler_params=pltpu.CompilerParams(use_tc_tiling_on_sc=False),
    )
    def k(y_hbm, src_hbm, dst_hbm, z_hbm, out_hbm, acc, src_v, dst_v,
          rows_v, sem):
        c = lax.axis_index("c")
        s = lax.axis_index("s")
        w = s * NC + c
        r0 = s * rows_per_s
        pltpu.sync_copy(z_hbm.at[pl.ds(r0, rows_per_s)],
                        acc.at[pl.ds(r0, rows_per_s)])
        pltpu.sync_copy(src_hbm.at[w], src_v)
        pltpu.sync_copy(dst_hbm.at[w], dst_v)
        plsc.subcore_barrier()

        @pl.loop(0, n_j)
        def edge_chunk(j):
            pltpu.async_copy(y_hbm.at[src_v.at[j]], rows_v, sem).wait()
            pltpu.sync_copy(rows_v, acc.at[dst_v.at[j]], add=True)

        plsc.subcore_barrier()
        pltpu.sync_copy(acc.at[pl.ds(r0, rows_per_s)],
                        out_hbm.at[c, pl.ds(r0, rows_per_s)])

    return k(y, src3, dst3, zeros)


EDGE_B0 = 128
EDGE_B1 = 128


def kernel(x, edge_index, W_self0, W_neigh0, b0, W_self1, W_neigh1, b1):
    n_edges = edge_index.shape[1]
    per_w = -(-n_edges // (NW * EDGE_B1)) * EDGE_B1
    e_pad = NW * per_w

    x_pad = jnp.zeros((N_PAD, x.shape[1]), jnp.float32).at[:N_NODES].set(x)
    n_tail = e_pad - n_edges
    tail_src = np.zeros((n_tail,), np.int32)
    tail_dst = N_NODES + (np.arange(n_tail, dtype=np.int32) % (N_PAD - N_NODES))
    src = jnp.concatenate([edge_index[0], tail_src])
    dst = jnp.concatenate([edge_index[1], tail_dst])
    src0 = src.reshape(NW, per_w // EDGE_B0, EDGE_B0)
    dst0 = dst.reshape(NW, per_w // EDGE_B0, EDGE_B0)
    src1 = src.reshape(NW, per_w // EDGE_B1, EDGE_B1)
    dst1 = dst.reshape(NW, per_w // EDGE_B1, EDGE_B1)

    d_hid = W_neigh0.shape[1]
    d_out = W_neigh1.shape[1]
    zeros_hid = jnp.zeros((N_PAD, d_hid), jnp.float32)
    zeros_out = jnp.zeros((N_PAD, d_out), jnp.float32)

    y0 = _tc_matmul(x_pad, W_neigh0)
    p0 = _sc_agg(y0, src0, dst0, zeros_hid, per_w // EDGE_B0, EDGE_B0)
    h, y1 = _tc_combine(x_pad, W_self0, p0, b0, relu=True, w_next=W_neigh1)
    p1 = _sc_agg(y1, src1, dst1, zeros_out, per_w // EDGE_B1, EDGE_B1)
    out = _tc_combine(h, W_self1, p1, b1, relu=False)
    return out[:N_NODES]


# speedup vs baseline: 1.9703x; 1.3214x over previous
import functools

import numpy as np

import jax
import jax.numpy as jnp
from jax import lax
from jax.experimental import pallas as pl
from jax.experimental.pallas import tpu as pltpu
from jax.experimental.pallas import tpu_sc as plsc

N_NODES = 10000
N_PAD = 10240
NC, NS = 2, 16
NW = NC * NS
DUMP_ROW = N_NODES + 64
ROW_BLK = 1024


def _tc_matmul(x, w):
    n, k = x.shape
    d = w.shape[1]

    def body(x_ref, w_ref, o_ref):
        o_ref[...] = jnp.dot(x_ref[...], w_ref[...],
                             preferred_element_type=jnp.float32)

    return pl.pallas_call(
        body,
        grid=(n // ROW_BLK,),
        in_specs=[
            pl.BlockSpec((ROW_BLK, k), lambda i: (i, 0)),
            pl.BlockSpec((k, d), lambda i: (0, 0)),
        ],
        out_specs=pl.BlockSpec((ROW_BLK, d), lambda i: (i, 0)),
        out_shape=jax.ShapeDtypeStruct((n, d), jnp.float32),
    )(x, w)


def _tc_combine(x, w_self, parts, b, relu, w_next=None):
    n, k = x.shape
    d = w_self.shape[1]
    d2 = None if w_next is None else w_next.shape[1]

    def body(x_ref, ws_ref, p_ref, b_ref, *rest):
        h = jnp.dot(x_ref[...], ws_ref[...], preferred_element_type=jnp.float32)
        h = h + p_ref[0] + p_ref[1] + b_ref[...]
        if relu:
            h = jnp.maximum(h, 0.0)
        if w_next is None:
            (o_ref,) = rest
            o_ref[...] = h
        else:
            wn_ref, h_ref, y_ref = rest
            h_ref[...] = h
            y_ref[...] = jnp.dot(h, wn_ref[...], preferred_element_type=jnp.float32)

    in_specs = [
        pl.BlockSpec((ROW_BLK, k), lambda i: (i, 0)),
        pl.BlockSpec((k, d), lambda i: (0, 0)),
        pl.BlockSpec((NC, ROW_BLK, d), lambda i: (0, i, 0)),
        pl.BlockSpec((1, d), lambda i: (0, 0)),
    ]
    operands = [x, w_self, parts, b.reshape(1, d)]
    if w_next is None:
        out_specs = pl.BlockSpec((ROW_BLK, d), lambda i: (i, 0))
        out_shape = jax.ShapeDtypeStruct((n, d), jnp.float32)
    else:
        in_specs.append(pl.BlockSpec((d, d2), lambda i: (0, 0)))
        operands.append(w_next)
        out_specs = [
            pl.BlockSpec((ROW_BLK, d), lambda i: (i, 0)),
            pl.BlockSpec((ROW_BLK, d2), lambda i: (i, 0)),
        ]
        out_shape = [
            jax.ShapeDtypeStruct((n, d), jnp.float32),
            jax.ShapeDtypeStruct((n, d2), jnp.float32),
        ]

    return pl.pallas_call(
        body,
        grid=(n // ROW_BLK,),
        in_specs=in_specs,
        out_specs=out_specs,
        out_shape=out_shape,
    )(*operands)


def _sc_agg(y, src2, dst2, zeros, edge_b, cnt_a, cnt_b, core_a):
    d = y.shape[1]
    n_max = max(cnt_a, cnt_b)
    rows_per_s = N_PAD // NS
    mesh = plsc.VectorSubcoreMesh(core_axis_name="c", subcore_axis_name="s")

    @functools.partial(
        pl.kernel,
        out_type=jax.ShapeDtypeStruct((NC, N_PAD, d), jnp.float32),
        mesh=mesh,
        scratch_types=[
            pltpu.VMEM_SHARED((N_PAD, d), jnp.float32),
            pltpu.VMEM((n_max, edge_b), jnp.int32),
            pltpu.VMEM((n_max, edge_b), jnp.int32),
            pltpu.VMEM((edge_b, d), jnp.float32),
            pltpu.SemaphoreType.DMA,
        ],
        compiler_params=pltpu.CompilerParams(use_tc_tiling_on_sc=False),
    )
    def k(y_hbm, src_hbm, dst_hbm, z_hbm, out_hbm, acc, src_v, dst_v,
          rows_v, sem):
        c = lax.axis_index("c")
        s = lax.axis_index("s")
        r0 = s * rows_per_s
        pltpu.sync_copy(z_hbm.at[pl.ds(r0, rows_per_s)],
                        acc.at[pl.ds(r0, rows_per_s)])

        @pl.when(c == core_a)
        def _():
            pltpu.sync_copy(src_hbm.at[pl.ds(s * cnt_a, cnt_a)],
                            src_v.at[pl.ds(0, cnt_a)])
            pltpu.sync_copy(dst_hbm.at[pl.ds(s * cnt_a, cnt_a)],
                            dst_v.at[pl.ds(0, cnt_a)])

        @pl.when(c != core_a)
        def _():
            pltpu.sync_copy(src_hbm.at[pl.ds(NS * cnt_a + s * cnt_b, cnt_b)],
                            src_v.at[pl.ds(0, cnt_b)])
            pltpu.sync_copy(dst_hbm.at[pl.ds(NS * cnt_a + s * cnt_b, cnt_b)],
                            dst_v.at[pl.ds(0, cnt_b)])

        plsc.subcore_barrier()
        cnt = jnp.where(c == core_a, cnt_a, cnt_b)

        @pl.loop(0, cnt)
        def edge_chunk(j):
            pltpu.async_copy(y_hbm.at[src_v.at[j]], rows_v, sem).wait()
            pltpu.sync_copy(rows_v, acc.at[dst_v.at[j]], add=True)

        plsc.subcore_barrier()
        pltpu.sync_copy(acc.at[pl.ds(r0, rows_per_s)],
                        out_hbm.at[c, pl.ds(r0, rows_per_s)])

    return k(y, src2, dst2, zeros)


EDGE_B = 128
CNT_A = 100
CNT_B = 57
CORE_A = 0


def kernel(x, edge_index, W_self0, W_neigh0, b0, W_self1, W_neigh1, b1):
    n_edges = edge_index.shape[1]
    e_pad = NS * (CNT_A + CNT_B) * EDGE_B
    assert e_pad >= n_edges

    x_pad = jnp.zeros((N_PAD, x.shape[1]), jnp.float32).at[:N_NODES].set(x)
    n_tail = e_pad - n_edges
    tail_src = np.zeros((n_tail,), np.int32)
    tail_dst = N_NODES + (np.arange(n_tail, dtype=np.int32) % (N_PAD - N_NODES))
    src2 = jnp.concatenate([edge_index[0], tail_src]).reshape(-1, EDGE_B)
    dst2 = jnp.concatenate([edge_index[1], tail_dst]).reshape(-1, EDGE_B)

    d_hid = W_neigh0.shape[1]
    d_out = W_neigh1.shape[1]
    zeros_hid = jnp.zeros((N_PAD, d_hid), jnp.float32)
    zeros_out = jnp.zeros((N_PAD, d_out), jnp.float32)

    y0 = _tc_matmul(x_pad, W_neigh0)
    p0 = _sc_agg(y0, src2, dst2, zeros_hid, EDGE_B, CNT_A, CNT_B, CORE_A)
    h, y1 = _tc_combine(x_pad, W_self0, p0, b0, relu=True, w_next=W_neigh1)
    p1 = _sc_agg(y1, src2, dst2, zeros_out, EDGE_B, CNT_A, CNT_B, CORE_A)
    out = _tc_combine(h, W_self1, p1, b1, relu=False)
    return out[:N_NODES]
